# Initial kernel scaffold; baseline (speedup 1.0000x reference)
import jax
import jax.numpy as jnp
from jax.experimental import pallas as pl


def kernel(x, edge_index, edge_weight, W0, b0, g0, be0, W1, b1, g1, be1, W2, b2):
    raise NotImplementedError("write your pallas kernel here")


# speedup vs baseline: 6.1533x; 6.1533x over previous
import functools

import jax
import jax.numpy as jnp
from jax import lax
from jax.experimental import pallas as pl
from jax.experimental.pallas import tpu as pltpu
from jax.experimental.pallas import tpu_sc as plsc

EPS = 1e-5
NC = 2
NS = 16
L = 16
CH = 128


def _zero_fill(ref, nrows, ncolgroups):
    zeros = jnp.zeros((L,), jnp.float32)

    def body(r, _):
        for cg in range(ncolgroups):
            ref[r, pl.ds(cg * L, L)] = zeros
        return 0

    lax.fori_loop(0, nrows, body, 0)


def _rows_per_tile(n_nodes):
    rpt = -(-n_nodes // NS)
    return ((rpt + 7) // 8) * 8


def _zero_acc(zbuf, acc_sh, base, rpt):
    nfull, rem = divmod(rpt, CH)
    for k in range(nfull):
        pltpu.sync_copy(zbuf.at[pl.ds(0, CH)],
                        acc_sh.at[pl.ds(base + k * CH, CH)])
    if rem:
        pltpu.sync_copy(zbuf.at[pl.ds(0, rem)],
                        acc_sh.at[pl.ds(base + nfull * CH, rem)])


def _make_deg_kernel(n_nodes, nch, d):
    mesh = plsc.VectorSubcoreMesh(core_axis_name="c", subcore_axis_name="s")
    rpt = _rows_per_tile(n_nodes)
    n_pad = rpt * NS

    @functools.partial(
        pl.kernel,
        mesh=mesh,
        out_type=jax.ShapeDtypeStruct((NC, n_pad, d), jnp.float32),
        scratch_types=[
            pltpu.VMEM((CH,), jnp.int32),
            pltpu.VMEM((CH, d), jnp.float32),
            pltpu.VMEM((CH, L), jnp.float32),
            pltpu.VMEM_SHARED((n_pad, d), jnp.float32),
        ],
    )
    def deg_kernel(dst_hbm, ewx_hbm, out_hbm, dbuf, rows_v, ebuf, acc_sh):
        c = lax.axis_index("c")
        s = lax.axis_index("s")
        w = s * NC + c
        base = s * rpt

        _zero_fill(rows_v, CH, d // L)
        _zero_acc(rows_v, acc_sh, base, rpt)
        plsc.subcore_barrier()

        def body(j, _):
            pltpu.sync_copy(dst_hbm.at[w, j], dbuf)
            pltpu.sync_copy(ewx_hbm.at[w, j], ebuf)

            def row(r, _):
                rows_v[r, pl.ds(0, L)] = ebuf[r]
                return 0

            lax.fori_loop(0, CH, row, 0)
            pltpu.sync_copy(rows_v, acc_sh.at[dbuf], add=True)
            return 0

        lax.fori_loop(0, nch, body, 0)
        plsc.subcore_barrier()
        pltpu.sync_copy(acc_sh.at[pl.ds(base, rpt)],
                        out_hbm.at[c, pl.ds(base, rpt)])

    return deg_kernel


def _make_scatter_kernel(n_nodes, nch, d):
    mesh = plsc.VectorSubcoreMesh(core_axis_name="c", subcore_axis_name="s")
    rpt = _rows_per_tile(n_nodes)
    n_pad = rpt * NS
    ncg = d // L

    @functools.partial(
        pl.kernel,
        mesh=mesh,
        out_type=jax.ShapeDtypeStruct((NC, n_pad, d), jnp.float32),
        scratch_types=[
            pltpu.VMEM((CH,), jnp.int32),
            pltpu.VMEM((CH,), jnp.int32),
            pltpu.VMEM((CH, d), jnp.float32),
            pltpu.VMEM((CH, L), jnp.float32),
            pltpu.VMEM_SHARED((n_pad, d), jnp.float32),
        ],
    )
    def scat_kernel(g_hbm, src_hbm, dst_hbm, ewx_hbm, out_hbm,
                    sbuf, dbuf, rows_v, ebuf, acc_sh):
        c = lax.axis_index("c")
        s = lax.axis_index("s")
        w = s * NC + c
        base = s * rpt

        _zero_fill(rows_v, CH, ncg)
        _zero_acc(rows_v, acc_sh, base, rpt)
        plsc.subcore_barrier()

        def chunk(j, _):
            pltpu.sync_copy(src_hbm.at[w, j], sbuf)
            pltpu.sync_copy(dst_hbm.at[w, j], dbuf)
            pltpu.sync_copy(ewx_hbm.at[w, j], ebuf)
            pltpu.sync_copy(g_hbm.at[sbuf], rows_v)

            def row(r, _):
                ewb = ebuf[r]
                for cg in range(ncg):
                    rows_v[r, pl.ds(cg * L, L)] = (
                        rows_v[r, pl.ds(cg * L, L)] * ewb)
                return 0

            lax.fori_loop(0, CH, row, 0)
            pltpu.sync_copy(rows_v, acc_sh.at[dbuf], add=True)
            return 0

        lax.fori_loop(0, nch, chunk, 0)
        plsc.subcore_barrier()
        pltpu.sync_copy(acc_sh.at[pl.ds(base, rpt)],
                        out_hbm.at[c, pl.ds(base, rpt)])

    return scat_kernel


def _tc_call(body, out_shapes):
    return pl.pallas_call(
        body,
        out_shape=[jax.ShapeDtypeStruct(s, jnp.float32) for s in out_shapes],
    )


def _tck1_body(n, x_ref, w_ref, degp_ref, dis_ref, g_ref):
    deg = degp_ref[0, 0:n, 0:1] + degp_ref[1, 0:n, 0:1] + 1.0
    dis = jnp.where(deg > 0, lax.rsqrt(deg), 0.0)
    dis_ref[...] = dis
    m = jnp.dot(x_ref[...], w_ref[...], preferred_element_type=jnp.float32)
    g_ref[...] = m * dis


def _tck_mid_body(n, s_ref, g_ref, dis_ref, b_ref, gam_ref, bet_ref, w_ref,
                  gnext_ref):
    dis = dis_ref[...]
    t = (s_ref[0, 0:n, :] + s_ref[1, 0:n, :] + g_ref[...]) * dis \
        + b_ref[...][None, :]
    mu = jnp.mean(t, axis=0, keepdims=True)
    var = jnp.mean((t - mu) ** 2, axis=0, keepdims=True)
    h = (t - mu) * lax.rsqrt(var + EPS) * gam_ref[...][None, :] \
        + bet_ref[...][None, :]
    h = jnp.maximum(h, 0.0)
    m = jnp.dot(h, w_ref[...], preferred_element_type=jnp.float32)
    gnext_ref[...] = m * dis


def _tck_final_body(n, d_out, s_ref, g_ref, dis_ref, b_ref, out_ref):
    o = (s_ref[0, 0:n, 0:d_out] + s_ref[1, 0:n, 0:d_out]
         + g_ref[..., 0:d_out]) * dis_ref[...] + b_ref[...][None, :]
    o = o - jnp.max(o, axis=-1, keepdims=True)
    out_ref[...] = o - jnp.log(jnp.sum(jnp.exp(o), axis=-1, keepdims=True))


def kernel(x, edge_index, edge_weight, W0, b0, g0, be0, W1, b1, g1, be1,
           W2, b2):
    n = x.shape[0]
    d_in = x.shape[1]
    d_h = W0.shape[1]
    d_out = W2.shape[1]
    e = edge_index.shape[1]

    nw = NC * NS
    nch = -(-e // (nw * CH))
    e_pad = nw * CH * nch
    pad = e_pad - e

    src = jnp.pad(edge_index[0], (0, pad))
    dst = jnp.pad(edge_index[1], (0, pad))
    ewp = jnp.pad(edge_weight, (0, pad))
    src3 = src.reshape(nw, nch, CH)
    dst3 = dst.reshape(nw, nch, CH)
    ewx = jnp.broadcast_to(ewp[:, None], (e_pad, L)).reshape(nw, nch, CH, L)

    degp = _make_deg_kernel(n, nch, d_h)(dst3, ewx)
    dis, gg0 = _tc_call(functools.partial(_tck1_body, n),
                        [(n, 1), (n, d_h)])(x, W0, degp)

    scat_h = _make_scatter_kernel(n, nch, d_h)
    mid = functools.partial(_tck_mid_body, n)
    fin = functools.partial(_tck_final_body, n, d_out)
    W2p = jnp.pad(W2, ((0, 0), (0, d_h - d_out)))

    s0 = scat_h(gg0, src3, dst3, ewx)
    (gg1,) = _tc_call(mid, [(n, d_h)])(s0, gg0, dis, b0, g0, be0, W1)
    s1 = scat_h(gg1, src3, dst3, ewx)
    (gg2,) = _tc_call(mid, [(n, d_h)])(s1, gg1, dis, b1, g1, be1, W2p)
    s2 = scat_h(gg2, src3, dst3, ewx)
    (out,) = _tc_call(fin, [(n, d_out)])(s2, gg2, dis, b2)
    return out
